# baseline (device time: 94305 ns/iter reference)
import jax
import jax.numpy as jnp
from jax import lax
from jax.experimental import pallas as pl
from jax.experimental.pallas import tpu as pltpu

N_DEV = 4
NSUB = 2


def _gelu(z):
    return 0.5 * z * (1.0 + jnp.tanh(0.7978845608 * (z + 0.044715 * z * z * z)))


def kernel(A, B):
    m, k = A.shape
    _, n = B.shape
    ch = m // N_DEV
    hf = ch // 2
    sub = hf // NSUB

    def body(a_ref, b_ref, out_ref, pbuf, b_buf,
             comm_cw, comm_ccw, cw_s, cw_r, ccw_s, ccw_r):
        my = lax.axis_index("i")
        left = (my + N_DEV - 1) % N_DEV
        right = (my + 1) % N_DEV

        b_buf[...] = b_ref[...].astype(jnp.bfloat16)

        barrier_sem = pltpu.get_barrier_semaphore()
        for nbr in (left, right):
            pl.semaphore_signal(
                barrier_sem, inc=1,
                device_id=(nbr,), device_id_type=pl.DeviceIdType.MESH,
            )
        pl.semaphore_wait(barrier_sem, 2)

        def compute_rows(off, rows):
            a_c = a_ref[pl.ds(off, rows), :].astype(jnp.bfloat16)
            p = lax.dot_general(
                a_c, b_buf[...], (((1,), (0,)), ((), ())),
                preferred_element_type=jnp.float32,
            )
            pbuf[pl.ds(off, rows), :] = p.astype(jnp.bfloat16)

        def rs_cw(h, s):
            off = ((my - h + N_DEV) % N_DEV) * ch + s * sub
            return pltpu.make_async_remote_copy(
                src_ref=pbuf.at[pl.ds(off, sub), :],
                dst_ref=comm_cw.at[h, pl.ds(s * sub, sub), :],
                send_sem=cw_s.at[NSUB * h + s],
                recv_sem=cw_r.at[NSUB * h + s],
                device_id=(right,), device_id_type=pl.DeviceIdType.MESH,
            )

        def rs_ccw(h, s):
            off = ((my + h) % N_DEV) * ch + hf + s * sub
            return pltpu.make_async_remote_copy(
                src_ref=pbuf.at[pl.ds(off, sub), :],
                dst_ref=comm_ccw.at[h, pl.ds(s * sub, sub), :],
                send_sem=ccw_s.at[NSUB * h + s],
                recv_sem=ccw_r.at[NSUB * h + s],
                device_id=(left,), device_id_type=pl.DeviceIdType.MESH,
            )

        def ag_cw(g, s):
            off = ((my + 1 - g + N_DEV) % N_DEV) * ch + s * sub
            return pltpu.make_async_remote_copy(
                src_ref=out_ref.at[pl.ds(off, sub), :],
                dst_ref=out_ref.at[pl.ds(off, sub), :],
                send_sem=cw_s.at[NSUB * (N_DEV - 1 + g) + s],
                recv_sem=cw_r.at[NSUB * (N_DEV - 1 + g) + s],
                device_id=(right,), device_id_type=pl.DeviceIdType.MESH,
            )

        def ag_ccw(g, s):
            off = ((my - 1 + g + N_DEV) % N_DEV) * ch + hf + s * sub
            return pltpu.make_async_remote_copy(
                src_ref=out_ref.at[pl.ds(off, sub), :],
                dst_ref=out_ref.at[pl.ds(off, sub), :],
                send_sem=ccw_s.at[NSUB * (N_DEV - 1 + g) + s],
                recv_sem=ccw_r.at[NSUB * (N_DEV - 1 + g) + s],
                device_id=(left,), device_id_type=pl.DeviceIdType.MESH,
            )

        compute_rows(my * ch, ch)
        rdmas_cw = {}
        rdmas_ccw = {}
        for d, start in ((rdmas_cw, rs_cw), (rdmas_ccw, rs_ccw)):
            for s in range(NSUB):
                d[(0, s)] = start(0, s)
                d[(0, s)].start()
        compute_rows(((my + 1) % N_DEV) * ch, ch)
        compute_rows(((my + 3) % N_DEV) * ch, ch)
        compute_rows(((my + 2) % N_DEV) * ch, ch)

        for h in range(N_DEV - 1):
            rc_cw = ((my - h - 1 + N_DEV) % N_DEV) * ch
            rc_ccw = ((my + h + 1) % N_DEV) * ch + hf
            for s in range(NSUB):
                rdmas_cw[(h, s)].wait()
                o = rc_cw + s * sub
                pbuf[pl.ds(o, sub), :] = (
                    pbuf[pl.ds(o, sub), :] + comm_cw[h, pl.ds(s * sub, sub), :]
                )
                if h < N_DEV - 2:
                    nxt = rs_cw(h + 1, s)
                    nxt.start()
                    rdmas_cw[(h + 1, s)] = nxt
                rdmas_ccw[(h, s)].wait()
                o2 = rc_ccw + s * sub
                pbuf[pl.ds(o2, sub), :] = (
                    pbuf[pl.ds(o2, sub), :] + comm_ccw[h, pl.ds(s * sub, sub), :]
                )
                if h < N_DEV - 2:
                    nxt = rs_ccw(h + 1, s)
                    nxt.start()
                    rdmas_ccw[(h + 1, s)] = nxt

        ol = ((my + 1) % N_DEV) * ch
        orr = ((my + N_DEV - 1) % N_DEV) * ch + hf
        ags_cw = {}
        ags_ccw = {}
        for s in range(NSUB):
            o = ol + s * sub
            gl = _gelu(pbuf[pl.ds(o, sub), :].astype(jnp.float32))
            out_ref[pl.ds(o, sub), :] = gl.astype(jnp.bfloat16)
            ags_cw[(0, s)] = ag_cw(0, s)
            ags_cw[(0, s)].start()
            o2 = orr + s * sub
            gr = _gelu(pbuf[pl.ds(o2, sub), :].astype(jnp.float32))
            out_ref[pl.ds(o2, sub), :] = gr.astype(jnp.bfloat16)
            ags_ccw[(0, s)] = ag_ccw(0, s)
            ags_ccw[(0, s)].start()

        for g in range(N_DEV - 1):
            for s in range(NSUB):
                ags_cw[(g, s)].wait()
                if g < N_DEV - 2:
                    nxt = ag_cw(g + 1, s)
                    nxt.start()
                    ags_cw[(g + 1, s)] = nxt
                ags_ccw[(g, s)].wait()
                if g < N_DEV - 2:
                    nxt = ag_ccw(g + 1, s)
                    nxt.start()
                    ags_ccw[(g + 1, s)] = nxt

    nsems = NSUB * 2 * (N_DEV - 1)
    return pl.pallas_call(
        body,
        out_shape=jax.ShapeDtypeStruct((m, n), jnp.bfloat16),
        in_specs=[
            pl.BlockSpec(memory_space=pltpu.VMEM),
            pl.BlockSpec(memory_space=pltpu.VMEM),
        ],
        out_specs=pl.BlockSpec(memory_space=pltpu.VMEM),
        scratch_shapes=[
            pltpu.VMEM((m, n), jnp.bfloat16),
            pltpu.VMEM((k, n), jnp.bfloat16),
            pltpu.VMEM((N_DEV - 1, hf, n), jnp.bfloat16),
            pltpu.VMEM((N_DEV - 1, hf, n), jnp.bfloat16),
            pltpu.SemaphoreType.DMA((nsems,)),
            pltpu.SemaphoreType.DMA((nsems,)),
            pltpu.SemaphoreType.DMA((nsems,)),
            pltpu.SemaphoreType.DMA((nsems,)),
        ],
        compiler_params=pltpu.CompilerParams(
            collective_id=0, vmem_limit_bytes=100 * 1024 * 1024
        ),
    )(A, B)


# device time: 92135 ns/iter; 1.0236x vs baseline; 1.0236x over previous
import jax
import jax.numpy as jnp
from jax import lax
from jax.experimental import pallas as pl
from jax.experimental.pallas import tpu as pltpu

N_DEV = 4
NPH = 2


def _gelu(z):
    return 0.5 * z * (1.0 + jnp.tanh(0.7978845608 * (z + 0.044715 * z * z * z)))


def kernel(A, B):
    m, k = A.shape
    _, n = B.shape
    ch = m // N_DEV
    hf = ch // 2
    hn = n // NPH

    def body(a_ref, b_ref, out_ref, pbuf, b_buf,
             comm_cw, comm_ccw, cw_s, cw_r, ccw_s, ccw_r):
        my = lax.axis_index("i")
        left = (my + N_DEV - 1) % N_DEV
        right = (my + 1) % N_DEV

        b_buf[...] = b_ref[...].astype(jnp.bfloat16)

        barrier_sem = pltpu.get_barrier_semaphore()
        for nbr in (left, right):
            pl.semaphore_signal(
                barrier_sem, inc=1,
                device_id=(nbr,), device_id_type=pl.DeviceIdType.MESH,
            )
        pl.semaphore_wait(barrier_sem, 2)

        def compute_rows(off, rows):
            a_c = a_ref[pl.ds(off, rows), :].astype(jnp.bfloat16)
            p = lax.dot_general(
                a_c, b_buf[...], (((1,), (0,)), ((), ())),
                preferred_element_type=jnp.float32,
            )
            pbuf[pl.ds(off, rows), :] = p.astype(jnp.bfloat16)

        def sem(ph, stage):
            return 6 * ph + stage

        def rs_cw(ph, h):
            off = ((my - h + N_DEV) % N_DEV) * ch
            return pltpu.make_async_remote_copy(
                src_ref=pbuf.at[pl.ds(off, hf), ph * hn:(ph + 1) * hn],
                dst_ref=comm_cw.at[ph, h],
                send_sem=cw_s.at[sem(ph, h)],
                recv_sem=cw_r.at[sem(ph, h)],
                device_id=(right,), device_id_type=pl.DeviceIdType.MESH,
            )

        def rs_ccw(ph, h):
            off = ((my + h) % N_DEV) * ch + hf
            return pltpu.make_async_remote_copy(
                src_ref=pbuf.at[pl.ds(off, hf), ph * hn:(ph + 1) * hn],
                dst_ref=comm_ccw.at[ph, h],
                send_sem=ccw_s.at[sem(ph, h)],
                recv_sem=ccw_r.at[sem(ph, h)],
                device_id=(left,), device_id_type=pl.DeviceIdType.MESH,
            )

        def ag_cw(ph, g):
            off = ((my + 1 - g + N_DEV) % N_DEV) * ch
            return pltpu.make_async_remote_copy(
                src_ref=out_ref.at[pl.ds(off, hf), ph * hn:(ph + 1) * hn],
                dst_ref=out_ref.at[pl.ds(off, hf), ph * hn:(ph + 1) * hn],
                send_sem=cw_s.at[sem(ph, 3 + g)],
                recv_sem=cw_r.at[sem(ph, 3 + g)],
                device_id=(right,), device_id_type=pl.DeviceIdType.MESH,
            )

        def ag_ccw(ph, g):
            off = ((my - 1 + g + N_DEV) % N_DEV) * ch + hf
            return pltpu.make_async_remote_copy(
                src_ref=out_ref.at[pl.ds(off, hf), ph * hn:(ph + 1) * hn],
                dst_ref=out_ref.at[pl.ds(off, hf), ph * hn:(ph + 1) * hn],
                send_sem=ccw_s.at[sem(ph, 3 + g)],
                recv_sem=ccw_r.at[sem(ph, 3 + g)],
                device_id=(left,), device_id_type=pl.DeviceIdType.MESH,
            )

        compute_rows(my * ch, ch)
        cw = {}
        ccw = {}
        for ph in range(NPH):
            cw[(ph, 0)] = rs_cw(ph, 0)
            cw[(ph, 0)].start()
            ccw[(ph, 0)] = rs_ccw(ph, 0)
            ccw[(ph, 0)].start()
        compute_rows(((my + 1) % N_DEV) * ch, ch)
        compute_rows(((my + 3) % N_DEV) * ch, ch)
        compute_rows(((my + 2) % N_DEV) * ch, ch)

        def rs_step(ph, h):
            cw[(ph, h)].wait()
            o = ((my - h - 1 + N_DEV) % N_DEV) * ch
            cols = slice(ph * hn, (ph + 1) * hn)
            pbuf[pl.ds(o, hf), cols] = (
                pbuf[pl.ds(o, hf), cols] + comm_cw[ph, h]
            )
            if h < N_DEV - 2:
                cw[(ph, h + 1)] = rs_cw(ph, h + 1)
                cw[(ph, h + 1)].start()
            ccw[(ph, h)].wait()
            o2 = ((my + h + 1) % N_DEV) * ch + hf
            pbuf[pl.ds(o2, hf), cols] = (
                pbuf[pl.ds(o2, hf), cols] + comm_ccw[ph, h]
            )
            if h < N_DEV - 2:
                ccw[(ph, h + 1)] = rs_ccw(ph, h + 1)
                ccw[(ph, h + 1)].start()

        for h in range(N_DEV - 2):
            for ph in range(NPH):
                rs_step(ph, h)

        ol = ((my + 1) % N_DEV) * ch
        orr = ((my + N_DEV - 1) % N_DEV) * ch + hf
        ag = {}
        agc = {}
        for ph in range(NPH):
            rs_step(ph, N_DEV - 2)
            cols = slice(ph * hn, (ph + 1) * hn)
            gl = _gelu(pbuf[pl.ds(ol, hf), cols].astype(jnp.float32))
            out_ref[pl.ds(ol, hf), cols] = gl.astype(jnp.bfloat16)
            ag[(ph, 0)] = ag_cw(ph, 0)
            ag[(ph, 0)].start()
            gr = _gelu(pbuf[pl.ds(orr, hf), cols].astype(jnp.float32))
            out_ref[pl.ds(orr, hf), cols] = gr.astype(jnp.bfloat16)
            agc[(ph, 0)] = ag_ccw(ph, 0)
            agc[(ph, 0)].start()

        for g in range(N_DEV - 1):
            for ph in range(NPH):
                ag[(ph, g)].wait()
                if g < N_DEV - 2:
                    ag[(ph, g + 1)] = ag_cw(ph, g + 1)
                    ag[(ph, g + 1)].start()
                agc[(ph, g)].wait()
                if g < N_DEV - 2:
                    agc[(ph, g + 1)] = ag_ccw(ph, g + 1)
                    agc[(ph, g + 1)].start()

    nsems = 6 * NPH
    return pl.pallas_call(
        body,
        out_shape=jax.ShapeDtypeStruct((m, n), jnp.bfloat16),
        in_specs=[
            pl.BlockSpec(memory_space=pltpu.VMEM),
            pl.BlockSpec(memory_space=pltpu.VMEM),
        ],
        out_specs=pl.BlockSpec(memory_space=pltpu.VMEM),
        scratch_shapes=[
            pltpu.VMEM((m, n), jnp.bfloat16),
            pltpu.VMEM((k, n), jnp.bfloat16),
            pltpu.VMEM((NPH, N_DEV - 1, hf, hn), jnp.bfloat16),
            pltpu.VMEM((NPH, N_DEV - 1, hf, hn), jnp.bfloat16),
            pltpu.SemaphoreType.DMA((nsems,)),
            pltpu.SemaphoreType.DMA((nsems,)),
            pltpu.SemaphoreType.DMA((nsems,)),
            pltpu.SemaphoreType.DMA((nsems,)),
        ],
        compiler_params=pltpu.CompilerParams(
            collective_id=0, vmem_limit_bytes=100 * 1024 * 1024
        ),
    )(A, B)
